# baseline (speedup 1.0000x reference)
import functools

import jax
import jax.numpy as jnp
from jax import lax
from jax.experimental import pallas as pl
from jax.experimental.pallas import tpu as pltpu
from jax.experimental.pallas import tpu_sc as plsc

DIGIT_DIM = 16
POS_DIM = 8
OUT_DIM = DIGIT_DIM + 2 * POS_DIM

BATCH = 16384
NPOS = 81
NTOK = BATCH * NPOS
NWORKERS = 32
TOK_PER_TILE = NTOK // NWORKERS
IDX_ROW = 128
GPB = 18
CHUNK = GPB * IDX_ROW
ITERS = TOK_PER_TILE // CHUNK


def _sc_gather(tab, idx):
    mesh = plsc.VectorSubcoreMesh(core_axis_name="c", subcore_axis_name="s")

    @functools.partial(
        pl.kernel,
        mesh=mesh,
        compiler_params=pltpu.CompilerParams(use_tc_tiling_on_sc=False),
        out_type=jax.ShapeDtypeStruct((NTOK, OUT_DIM), jnp.float32),
        scratch_types=[
            pltpu.VMEM((TOK_PER_TILE,), jnp.int32),
            pltpu.VMEM((CHUNK, OUT_DIM), jnp.float32),
            pltpu.SemaphoreType.DMA,
        ],
    )
    def body(tab_hbm, idx_hbm, out_hbm, idx_v, buf_v, sem):
        wid = lax.axis_index("s") * 2 + lax.axis_index("c")
        t0 = wid * TOK_PER_TILE
        pltpu.sync_copy(idx_hbm.at[pl.ds(t0, TOK_PER_TILE)], idx_v)

        def step(i, carry):
            cps = [
                pltpu.async_copy(
                    tab_hbm.at[idx_v.at[pl.ds((i * GPB + j) * IDX_ROW, IDX_ROW)]],
                    buf_v.at[pl.ds(j * IDX_ROW, IDX_ROW)],
                    sem)
                for j in range(GPB)
            ]
            for cp in cps:
                cp.wait()
            pltpu.sync_copy(buf_v, out_hbm.at[pl.ds(t0 + i * CHUNK, CHUNK)])
            return carry

        lax.fori_loop(0, ITERS, step, 0)

    return body(tab, idx)


def kernel(x, digit_emb, row_emb, col_emb):
    pos = jnp.arange(NPOS, dtype=jnp.int32)
    posemb = jnp.concatenate(
        [jnp.take(row_emb, pos // 9, axis=0),
         jnp.take(col_emb, pos % 9, axis=0)], axis=-1)
    tab = jnp.concatenate(
        [jnp.broadcast_to(digit_emb[None, :, :], (NPOS, 10, DIGIT_DIM)),
         jnp.broadcast_to(posemb[:, None, :], (NPOS, 10, 2 * POS_DIM))],
        axis=-1).reshape(NPOS * 10, OUT_DIM)
    idx = (x.astype(jnp.int32) + pos[None, :] * 10).reshape(NTOK)
    out = _sc_gather(tab, idx)
    return out.reshape(BATCH, NPOS, OUT_DIM)


# speedup vs baseline: 4.5811x; 4.5811x over previous
import functools

import jax
import jax.numpy as jnp
from jax import lax
from jax.experimental import pallas as pl
from jax.experimental.pallas import tpu as pltpu
from jax.experimental.pallas import tpu_sc as plsc

DIGIT_DIM = 16
POS_DIM = 8
OUT_DIM = DIGIT_DIM + 2 * POS_DIM

BATCH = 16384
NPOS = 81
NWORKERS = 32
BOARDS_PER_TILE = BATCH // NWORKERS
GPB = 16
NCHUNK = BOARDS_PER_TILE // GPB
NBODY = NCHUNK // 2


def _sc_gather(tab, idx):
    mesh = plsc.VectorSubcoreMesh(core_axis_name="c", subcore_axis_name="s")

    @functools.partial(
        pl.kernel,
        mesh=mesh,
        compiler_params=pltpu.CompilerParams(use_tc_tiling_on_sc=False),
        out_type=jax.ShapeDtypeStruct((BATCH, NPOS, OUT_DIM), jnp.float32),
        scratch_types=[
            pltpu.VMEM((BOARDS_PER_TILE, NPOS), jnp.int32),
            pltpu.VMEM((GPB, NPOS, OUT_DIM), jnp.float32),
            pltpu.VMEM((GPB, NPOS, OUT_DIM), jnp.float32),
            pltpu.SemaphoreType.DMA,
            pltpu.SemaphoreType.DMA,
            pltpu.SemaphoreType.DMA,
            pltpu.SemaphoreType.DMA,
        ],
    )
    def body(tab_hbm, idx_hbm, out_hbm, idx_v, buf0, buf1, gs0, gs1, ws0, ws1):
        wid = lax.axis_index("s") * 2 + lax.axis_index("c")
        b0 = wid * BOARDS_PER_TILE
        pltpu.sync_copy(idx_hbm.at[pl.ds(b0, BOARDS_PER_TILE)], idx_v)

        dummy = out_hbm.at[pl.ds(0, GPB)]

        def fire(chunk, buf, gsem):
            for j in range(GPB):
                pltpu.async_copy(
                    tab_hbm.at[idx_v.at[chunk * GPB + j]], buf.at[j], gsem)

        def wait_gathers(buf, gsem):
            pltpu.make_async_copy(dummy, buf, gsem).wait()

        def write(chunk, buf, wsem):
            pltpu.async_copy(
                buf, out_hbm.at[pl.ds(b0 + chunk * GPB, GPB)], wsem)

        def wait_write(buf, wsem):
            pltpu.make_async_copy(buf, dummy, wsem).wait()

        fire(0, buf0, gs0)

        def step(g, carry):
            a = 2 * g
            wait_gathers(buf0, gs0)

            @pl.when(g != 0)
            def _():
                wait_write(buf1, ws1)

            fire(a + 1, buf1, gs1)
            write(a, buf0, ws0)
            wait_gathers(buf1, gs1)
            wait_write(buf0, ws0)

            @pl.when(g != NBODY - 1)
            def _():
                fire(a + 2, buf0, gs0)

            write(a + 1, buf1, ws1)
            return carry

        lax.fori_loop(0, NBODY, step, 0)
        wait_write(buf1, ws1)

    return body(tab, idx)


def kernel(x, digit_emb, row_emb, col_emb):
    pos = jnp.arange(NPOS, dtype=jnp.int32)
    posemb = jnp.concatenate(
        [jnp.take(row_emb, pos // 9, axis=0),
         jnp.take(col_emb, pos % 9, axis=0)], axis=-1)
    tab = jnp.concatenate(
        [jnp.broadcast_to(digit_emb[None, :, :], (NPOS, 10, DIGIT_DIM)),
         jnp.broadcast_to(posemb[:, None, :], (NPOS, 10, 2 * POS_DIM))],
        axis=-1).reshape(NPOS * 10, OUT_DIM)
    idx = x.astype(jnp.int32) + pos[None, :] * 10
    return _sc_gather(tab, idx)


# speedup vs baseline: 4.8885x; 1.0671x over previous
import functools

import jax
import jax.numpy as jnp
from jax import lax
from jax.experimental import pallas as pl
from jax.experimental.pallas import tpu as pltpu
from jax.experimental.pallas import tpu_sc as plsc

DIGIT_DIM = 16
POS_DIM = 8
OUT_DIM = DIGIT_DIM + 2 * POS_DIM

BATCH = 16384
NPOS = 81
NSPLIT = 4
SBATCH = BATCH // NSPLIT
NWORKERS = 32
BOARDS_PER_TILE = SBATCH // NWORKERS
GPB = 8
NCHUNK = BOARDS_PER_TILE // GPB
NBODY = NCHUNK // 2


def _sc_gather(tab, idx):
    mesh = plsc.VectorSubcoreMesh(core_axis_name="c", subcore_axis_name="s")

    @functools.partial(
        pl.kernel,
        mesh=mesh,
        compiler_params=pltpu.CompilerParams(use_tc_tiling_on_sc=False),
        out_type=jax.ShapeDtypeStruct((SBATCH, NPOS, OUT_DIM), jnp.float32),
        scratch_types=[
            pltpu.VMEM((BOARDS_PER_TILE, NPOS), jnp.int32),
            pltpu.VMEM((GPB, NPOS, OUT_DIM), jnp.float32),
            pltpu.VMEM((GPB, NPOS, OUT_DIM), jnp.float32),
            pltpu.SemaphoreType.DMA,
            pltpu.SemaphoreType.DMA,
            pltpu.SemaphoreType.DMA,
            pltpu.SemaphoreType.DMA,
        ],
    )
    def body(tab_hbm, idx_hbm, out_hbm, idx_v, buf0, buf1, gs0, gs1, ws0, ws1):
        wid = lax.axis_index("s") * 2 + lax.axis_index("c")
        b0 = wid * BOARDS_PER_TILE
        pltpu.sync_copy(idx_hbm.at[pl.ds(b0, BOARDS_PER_TILE)], idx_v)

        dummy = out_hbm.at[pl.ds(0, GPB)]

        def fire(chunk, buf, gsem):
            for j in range(GPB):
                pltpu.async_copy(
                    tab_hbm.at[idx_v.at[chunk * GPB + j]], buf.at[j], gsem)

        def wait_gathers(buf, gsem):
            pltpu.make_async_copy(dummy, buf, gsem).wait()

        def write(chunk, buf, wsem):
            pltpu.async_copy(
                buf, out_hbm.at[pl.ds(b0 + chunk * GPB, GPB)], wsem)

        def wait_write(buf, wsem):
            pltpu.make_async_copy(buf, dummy, wsem).wait()

        fire(0, buf0, gs0)

        def step(g, carry):
            a = 2 * g
            wait_gathers(buf0, gs0)

            @pl.when(g != 0)
            def _():
                wait_write(buf1, ws1)

            fire(a + 1, buf1, gs1)
            write(a, buf0, ws0)
            wait_gathers(buf1, gs1)
            wait_write(buf0, ws0)

            @pl.when(g != NBODY - 1)
            def _():
                fire(a + 2, buf0, gs0)

            write(a + 1, buf1, ws1)
            return carry

        lax.fori_loop(0, NBODY, step, 0)
        wait_write(buf1, ws1)

    return body(tab, idx)


def kernel(x, digit_emb, row_emb, col_emb):
    pos = jnp.arange(NPOS, dtype=jnp.int32)
    posemb = jnp.concatenate(
        [jnp.take(row_emb, pos // 9, axis=0),
         jnp.take(col_emb, pos % 9, axis=0)], axis=-1)
    tab = jnp.concatenate(
        [jnp.broadcast_to(digit_emb[None, :, :], (NPOS, 10, DIGIT_DIM)),
         jnp.broadcast_to(posemb[:, None, :], (NPOS, 10, 2 * POS_DIM))],
        axis=-1).reshape(NPOS * 10, OUT_DIM)
    idx = x.astype(jnp.int32) + pos[None, :] * 10
    outs = [_sc_gather(tab, idx[c * SBATCH:(c + 1) * SBATCH])
            for c in range(NSPLIT)]
    return jnp.concatenate(outs, axis=0)
